# baseline (speedup 1.0000x reference)
import functools

import jax
import jax.numpy as jnp
from jax import lax
from jax.experimental import pallas as pl
from jax.experimental.pallas import tpu as pltpu
from jax.experimental.pallas import tpu_sc as plsc

_PH = (768, 1536)


def _level_body(resid_ref, w_ref, b2_ref, a2_ref, idx_ref, *, K):
    resid = resid_ref[...]
    w = w_ref[...]
    rb = resid.astype(jnp.bfloat16)
    wb = w.astype(jnp.bfloat16)
    conv = jax.lax.dot_general(
        rb, wb, (((1,), (1,)), ((), ())),
        preferred_element_type=jnp.float32)
    a2 = a2_ref[...]
    b2 = b2_ref[...]
    s = jnp.maximum((a2 + b2) - 2.0 * conv, 0.0)
    dd = s * jax.lax.rsqrt(s)
    R = resid.shape[0]
    iota = jax.lax.broadcasted_iota(jnp.int32, (R, K), 1)
    acc_v = None
    acc_i = None
    for lo, hi in ((0, _PH[0]), (_PH[0], _PH[1]), (_PH[1], K)):
        seg = dd[:, lo:hi]
        m = jnp.min(seg, axis=1, keepdims=True)
        si = jnp.min(jnp.where(seg == m, iota[:, lo:hi], K),
                     axis=1, keepdims=True)
        mb = m.astype(jnp.bfloat16).astype(jnp.float32)
        if acc_v is None:
            acc_v, acc_i = mb, si
        else:
            upd = m < acc_v
            acc_i = jnp.where(upd, si, acc_i)
            acc_v = jnp.where(upd, mb, acc_v)
    idx_ref[...] = acc_i


def _level_indices(resid, w, b2, a2, *, rows):
    N, D = resid.shape
    K = w.shape[0]
    grid = (N // rows,)
    idx = pl.pallas_call(
        functools.partial(_level_body, K=K),
        grid=grid,
        in_specs=[
            pl.BlockSpec((rows, D), lambda i: (i, 0)),
            pl.BlockSpec((K, D), lambda i: (0, 0)),
            pl.BlockSpec((1, K), lambda i: (0, 0)),
            pl.BlockSpec((rows, 1), lambda i: (i, 0)),
        ],
        out_specs=pl.BlockSpec((rows, 1), lambda i: (i, 0)),
        out_shape=jax.ShapeDtypeStruct((N, 1), jnp.int32),
        compiler_params=pltpu.CompilerParams(
            dimension_semantics=("arbitrary",)),
    )(resid, w, b2, a2)
    return idx


def _make_sc_gather(V, D, B):
    info = plsc.get_sparse_core_info()
    NC, NS = info.num_cores, info.num_subcores
    NW = NC * NS
    b_per_w = B // NW
    n_chunks = max(1, b_per_w // 192)
    ch = b_per_w // n_chunks
    mesh = plsc.VectorSubcoreMesh(core_axis_name="c", subcore_axis_name="s")

    @functools.partial(
        pl.kernel, mesh=mesh,
        out_type=jax.ShapeDtypeStruct((B, D), jnp.float32),
        scratch_types=[
            pltpu.VMEM((ch,), jnp.int32),
            pltpu.VMEM((ch, D), jnp.float32),
            pltpu.SemaphoreType.DMA,
        ],
    )
    def k(table_hbm, idx_hbm, out_hbm, idx_v, rows_v, sem):
        wid = lax.axis_index("s") * NC + lax.axis_index("c")
        base = wid * b_per_w
        for c in range(n_chunks):
            off = base + c * ch
            pltpu.sync_copy(idx_hbm.at[pl.ds(off, ch)], idx_v)
            pltpu.async_copy(table_hbm.at[idx_v], rows_v, sem).wait()
            pltpu.sync_copy(rows_v, out_hbm.at[pl.ds(off, ch)])

    return k


def kernel(embeddings, codebooks):
    B_, T_, D_ = embeddings.shape
    n_cb, K, _ = codebooks.shape
    N = B_ * T_
    rows = 600
    G = 2
    NH = N // G
    NPADH = ((NH + 1023) // 1024) * 1024
    gather = _make_sc_gather(K, D_, NPADH)
    residual = embeddings
    qs = []
    codes_l = []
    total_loss = jnp.float32(0.0)
    for l in range(n_cb):
        w = codebooks[l]
        b2 = jnp.sum(w * w, axis=1).reshape(1, K)
        a2f = jnp.sum(residual * residual, axis=2).reshape(N, 1)
        r2d = residual.reshape(N, D_)
        idx_g, q_g = [], []
        for g in range(G):
            sl = slice(g * NH, (g + 1) * NH)
            idx = _level_indices(r2d[sl], w, b2, a2f[sl], rows=rows)
            idx_g.append(idx.reshape(NH))
            idxp = jnp.pad(idx_g[g], (0, NPADH - NH))
            q_g.append(gather(w, idxp)[:NH])
        codes_l.append(jnp.concatenate(idx_g, axis=0))
        qfull = jnp.concatenate(q_g, axis=0).reshape(B_, T_, D_)
        qs.append(qfull)
        total_loss = total_loss + jnp.mean(
            (jax.lax.stop_gradient(residual) - qfull) ** 2)
        residual = residual - jax.lax.stop_gradient(qfull)
    codes = jnp.stack(codes_l, axis=-1).reshape(B_, T_, n_cb)
    quantized = qs[0]
    for l in range(1, n_cb):
        quantized = quantized + qs[l]
    quantized = embeddings + jax.lax.stop_gradient(quantized - embeddings)
    return codes, quantized, total_loss / n_cb


# speedup vs baseline: 1.1960x; 1.1960x over previous
import functools

import jax
import jax.numpy as jnp
from jax import lax
from jax.experimental import pallas as pl
from jax.experimental.pallas import tpu as pltpu
from jax.experimental.pallas import tpu_sc as plsc

_PH = (768, 1536)


def _level_body(resid_ref, w_ref, b2_ref, a2_ref, idx_ref, *, K):
    resid = resid_ref[...]
    w = w_ref[...]
    rb = resid.astype(jnp.bfloat16)
    wb = w.astype(jnp.bfloat16)
    conv = jax.lax.dot_general(
        rb, wb, (((1,), (1,)), ((), ())),
        preferred_element_type=jnp.float32)
    a2 = a2_ref[...]
    b2 = b2_ref[...]
    s = jnp.maximum((a2 + b2) - 2.0 * conv, 0.0)
    dd = s * jax.lax.rsqrt(s)
    R = resid.shape[0]
    iota = jax.lax.broadcasted_iota(jnp.int32, (R, K), 1)
    acc_v = None
    acc_i = None
    for lo, hi in ((0, _PH[0]), (_PH[0], _PH[1]), (_PH[1], K)):
        seg = dd[:, lo:hi]
        m = jnp.min(seg, axis=1, keepdims=True)
        si = jnp.min(jnp.where(seg == m, iota[:, lo:hi], K),
                     axis=1, keepdims=True)
        mb = m.astype(jnp.bfloat16).astype(jnp.float32)
        if acc_v is None:
            acc_v, acc_i = mb, si
        else:
            upd = m < acc_v
            acc_i = jnp.where(upd, si, acc_i)
            acc_v = jnp.where(upd, mb, acc_v)
    idx_ref[...] = acc_i


def _level_indices(resid, w, b2, a2, *, rows):
    N, D = resid.shape
    K = w.shape[0]
    grid = (N // rows,)
    idx = pl.pallas_call(
        functools.partial(_level_body, K=K),
        grid=grid,
        in_specs=[
            pl.BlockSpec((rows, D), lambda i: (i, 0)),
            pl.BlockSpec((K, D), lambda i: (0, 0)),
            pl.BlockSpec((1, K), lambda i: (0, 0)),
            pl.BlockSpec((rows, 1), lambda i: (i, 0)),
        ],
        out_specs=pl.BlockSpec((rows, 1), lambda i: (i, 0)),
        out_shape=jax.ShapeDtypeStruct((N, 1), jnp.int32),
        compiler_params=pltpu.CompilerParams(
            dimension_semantics=("arbitrary",)),
    )(resid, w, b2, a2)
    return idx


def _make_sc_gather(V, D, B):
    info = plsc.get_sparse_core_info()
    NC, NS = info.num_cores, info.num_subcores
    NW = NC * NS
    b_per_w = B // NW
    n_chunks = max(1, b_per_w // 192)
    ch = b_per_w // n_chunks
    mesh = plsc.VectorSubcoreMesh(core_axis_name="c", subcore_axis_name="s")

    @functools.partial(
        pl.kernel, mesh=mesh,
        out_type=jax.ShapeDtypeStruct((B, D), jnp.float32),
        scratch_types=[
            pltpu.VMEM((ch,), jnp.int32),
            pltpu.VMEM((ch, D), jnp.float32),
            pltpu.SemaphoreType.DMA,
        ],
    )
    def k(table_hbm, idx_hbm, out_hbm, idx_v, rows_v, sem):
        wid = lax.axis_index("s") * NC + lax.axis_index("c")
        base = wid * b_per_w
        for c in range(n_chunks):
            off = base + c * ch
            pltpu.sync_copy(idx_hbm.at[pl.ds(off, ch)], idx_v)
            pltpu.async_copy(table_hbm.at[idx_v], rows_v, sem).wait()
            pltpu.sync_copy(rows_v, out_hbm.at[pl.ds(off, ch)])

    return k


def kernel(embeddings, codebooks):
    B_, T_, D_ = embeddings.shape
    n_cb, K, _ = codebooks.shape
    N = B_ * T_
    rows = 600
    G = 1
    NH = N // G
    NPADH = ((NH + 1023) // 1024) * 1024
    gather = _make_sc_gather(K, D_, NPADH)
    residual = embeddings
    qs = []
    codes_l = []
    total_loss = jnp.float32(0.0)
    for l in range(n_cb):
        w = codebooks[l]
        b2 = jnp.sum(w * w, axis=1).reshape(1, K)
        a2f = jnp.sum(residual * residual, axis=2).reshape(N, 1)
        r2d = residual.reshape(N, D_)
        idx_g, q_g = [], []
        for g in range(G):
            sl = slice(g * NH, (g + 1) * NH)
            idx = _level_indices(r2d[sl], w, b2, a2f[sl], rows=rows)
            idx_g.append(idx.reshape(NH))
            idxp = jnp.pad(idx_g[g], (0, NPADH - NH))
            q_g.append(gather(w, idxp)[:NH])
        codes_l.append(jnp.concatenate(idx_g, axis=0))
        qfull = jnp.concatenate(q_g, axis=0).reshape(B_, T_, D_)
        qs.append(qfull)
        total_loss = total_loss + jnp.mean(
            (jax.lax.stop_gradient(residual) - qfull) ** 2)
        residual = residual - jax.lax.stop_gradient(qfull)
    codes = jnp.stack(codes_l, axis=-1).reshape(B_, T_, n_cb)
    quantized = qs[0]
    for l in range(1, n_cb):
        quantized = quantized + qs[l]
    quantized = embeddings + jax.lax.stop_gradient(quantized - embeddings)
    return codes, quantized, total_loss / n_cb


# speedup vs baseline: 1.2180x; 1.0184x over previous
import functools

import jax
import jax.numpy as jnp
from jax import lax
from jax.experimental import pallas as pl
from jax.experimental.pallas import tpu as pltpu
from jax.experimental.pallas import tpu_sc as plsc

_PH = (768, 1536)


def _level_body(resid_ref, w_ref, b2_ref, a2_ref, idx_ref, *, K):
    resid = resid_ref[...]
    w = w_ref[...]
    rb = resid.astype(jnp.bfloat16)
    wb = w.astype(jnp.bfloat16)
    conv = jax.lax.dot_general(
        rb, wb, (((1,), (1,)), ((), ())),
        preferred_element_type=jnp.float32)
    a2 = a2_ref[...]
    b2 = b2_ref[...]
    s = jnp.maximum((a2 + b2) - 2.0 * conv, 0.0)
    dd = s * jax.lax.rsqrt(s)
    R = resid.shape[0]
    iota = jax.lax.broadcasted_iota(jnp.int32, (R, K), 1)
    acc_v = None
    acc_i = None
    for lo, hi in ((0, _PH[0]), (_PH[0], _PH[1]), (_PH[1], K)):
        seg = dd[:, lo:hi]
        m = jnp.min(seg, axis=1, keepdims=True)
        si = jnp.min(jnp.where(seg == m, iota[:, lo:hi], K),
                     axis=1, keepdims=True)
        mb = m.astype(jnp.bfloat16).astype(jnp.float32)
        if acc_v is None:
            acc_v, acc_i = mb, si
        else:
            upd = m < acc_v
            acc_i = jnp.where(upd, si, acc_i)
            acc_v = jnp.where(upd, mb, acc_v)
    idx_ref[...] = acc_i


def _level_indices(resid, w, b2, a2, *, rows):
    N, D = resid.shape
    K = w.shape[0]
    grid = (N // rows,)
    idx = pl.pallas_call(
        functools.partial(_level_body, K=K),
        grid=grid,
        in_specs=[
            pl.BlockSpec((rows, D), lambda i: (i, 0)),
            pl.BlockSpec((K, D), lambda i: (0, 0)),
            pl.BlockSpec((1, K), lambda i: (0, 0)),
            pl.BlockSpec((rows, 1), lambda i: (i, 0)),
        ],
        out_specs=pl.BlockSpec((rows, 1), lambda i: (i, 0)),
        out_shape=jax.ShapeDtypeStruct((N, 1), jnp.int32),
        compiler_params=pltpu.CompilerParams(
            dimension_semantics=("arbitrary",)),
    )(resid, w, b2, a2)
    return idx


def _make_sc_gather(V, D, B):
    info = plsc.get_sparse_core_info()
    NC, NS = info.num_cores, info.num_subcores
    NW = NC * NS
    b_per_w = B // NW
    n_chunks = max(1, b_per_w // 192)
    ch = b_per_w // n_chunks
    mesh = plsc.VectorSubcoreMesh(core_axis_name="c", subcore_axis_name="s")

    @functools.partial(
        pl.kernel, mesh=mesh,
        out_type=jax.ShapeDtypeStruct((B, D), jnp.float32),
        scratch_types=[
            pltpu.VMEM((ch,), jnp.int32),
            pltpu.VMEM((ch, D), jnp.float32),
            pltpu.SemaphoreType.DMA,
        ],
    )
    def k(table_hbm, idx_hbm, out_hbm, idx_v, rows_v, sem):
        wid = lax.axis_index("s") * NC + lax.axis_index("c")
        base = wid * b_per_w
        for c in range(n_chunks):
            off = base + c * ch
            pltpu.sync_copy(idx_hbm.at[pl.ds(off, ch)], idx_v)
            pltpu.async_copy(table_hbm.at[idx_v], rows_v, sem).wait()
            pltpu.sync_copy(rows_v, out_hbm.at[pl.ds(off, ch)])

    return k


def kernel(embeddings, codebooks):
    B_, T_, D_ = embeddings.shape
    n_cb, K, _ = codebooks.shape
    N = B_ * T_
    rows = 1200
    G = 1
    NH = N // G
    NPADH = ((NH + 1023) // 1024) * 1024
    gather = _make_sc_gather(K, D_, NPADH)
    residual = embeddings
    qs = []
    codes_l = []
    total_loss = jnp.float32(0.0)
    for l in range(n_cb):
        w = codebooks[l]
        b2 = jnp.sum(w * w, axis=1).reshape(1, K)
        a2f = jnp.sum(residual * residual, axis=2).reshape(N, 1)
        r2d = residual.reshape(N, D_)
        idx_g, q_g = [], []
        for g in range(G):
            sl = slice(g * NH, (g + 1) * NH)
            rg = r2d if G == 1 else r2d[sl]
            ag = a2f if G == 1 else a2f[sl]
            idx = _level_indices(rg, w, b2, ag, rows=rows)
            idx_g.append(idx.reshape(NH))
            idxp = jnp.pad(idx_g[g], (0, NPADH - NH))
            q_g.append(gather(w, idxp)[:NH])
        codes_l.append(idx_g[0] if G == 1 else jnp.concatenate(idx_g, axis=0))
        qfull = (q_g[0] if G == 1
                 else jnp.concatenate(q_g, axis=0)).reshape(B_, T_, D_)
        qs.append(qfull)
        total_loss = total_loss + jnp.mean(
            (jax.lax.stop_gradient(residual) - qfull) ** 2)
        residual = residual - jax.lax.stop_gradient(qfull)
    codes = jnp.stack(codes_l, axis=-1).reshape(B_, T_, n_cb)
    quantized = qs[0]
    for l in range(1, n_cb):
        quantized = quantized + qs[l]
    quantized = embeddings + jax.lax.stop_gradient(quantized - embeddings)
    return codes, quantized, total_loss / n_cb


# speedup vs baseline: 1.2432x; 1.0207x over previous
import functools

import jax
import jax.numpy as jnp
from jax import lax
from jax.experimental import pallas as pl
from jax.experimental.pallas import tpu as pltpu
from jax.experimental.pallas import tpu_sc as plsc

_PH = (768, 1536)


def _level_body(resid_ref, w_ref, b2_ref, a2_ref, idx_ref, *, K):
    resid = resid_ref[...]
    w = w_ref[...]
    rb = resid.astype(jnp.bfloat16)
    wb = w.astype(jnp.bfloat16)
    conv = jax.lax.dot_general(
        rb, wb, (((1,), (1,)), ((), ())),
        preferred_element_type=jnp.float32)
    a2 = a2_ref[...]
    b2 = b2_ref[...]
    s = jnp.maximum((a2 + b2) - 2.0 * conv, 0.0)
    dd = s * jax.lax.rsqrt(s)
    R = resid.shape[0]
    iota = jax.lax.broadcasted_iota(jnp.int32, (R, K), 1)
    acc_v = None
    acc_i = None
    for lo, hi in ((0, _PH[0]), (_PH[0], _PH[1]), (_PH[1], K)):
        seg = dd[:, lo:hi]
        m = jnp.min(seg, axis=1, keepdims=True)
        si = jnp.min(jnp.where(seg == m, iota[:, lo:hi], K),
                     axis=1, keepdims=True)
        mb = m.astype(jnp.bfloat16).astype(jnp.float32)
        if acc_v is None:
            acc_v, acc_i = mb, si
        else:
            upd = m < acc_v
            acc_i = jnp.where(upd, si, acc_i)
            acc_v = jnp.where(upd, mb, acc_v)
    idx_ref[...] = acc_i


def _level_indices(resid, w, b2, a2, *, rows):
    N, D = resid.shape
    K = w.shape[0]
    grid = (N // rows,)
    idx = pl.pallas_call(
        functools.partial(_level_body, K=K),
        grid=grid,
        in_specs=[
            pl.BlockSpec((rows, D), lambda i: (i, 0)),
            pl.BlockSpec((K, D), lambda i: (0, 0)),
            pl.BlockSpec((1, K), lambda i: (0, 0)),
            pl.BlockSpec((rows, 1), lambda i: (i, 0)),
        ],
        out_specs=pl.BlockSpec((rows, 1), lambda i: (i, 0)),
        out_shape=jax.ShapeDtypeStruct((N, 1), jnp.int32),
        compiler_params=pltpu.CompilerParams(
            dimension_semantics=("arbitrary",)),
    )(resid, w, b2, a2)
    return idx


def _make_sc_gather(V, D, B):
    info = plsc.get_sparse_core_info()
    NC, NS = info.num_cores, info.num_subcores
    NW = NC * NS
    b_per_w = B // NW
    n_chunks = max(1, b_per_w // 96)
    ch = b_per_w // n_chunks
    mesh = plsc.VectorSubcoreMesh(core_axis_name="c", subcore_axis_name="s")

    @functools.partial(
        pl.kernel, mesh=mesh,
        out_type=jax.ShapeDtypeStruct((B, D), jnp.float32),
        scratch_types=[
            pltpu.VMEM((ch,), jnp.int32),
            pltpu.VMEM((ch,), jnp.int32),
            pltpu.VMEM((ch, D), jnp.float32),
            pltpu.VMEM((ch, D), jnp.float32),
            pltpu.SemaphoreType.DMA,
            pltpu.SemaphoreType.DMA,
        ],
    )
    def k(table_hbm, idx_hbm, out_hbm, i0, i1, r0, r1, s0, s1):
        wid = lax.axis_index("s") * NC + lax.axis_index("c")
        base = wid * b_per_w
        ibuf, rbuf, sem = (i0, i1), (r0, r1), (s0, s1)
        pltpu.sync_copy(idx_hbm.at[pl.ds(base, ch)], i0)
        pending = pltpu.async_copy(table_hbm.at[i0], r0, s0)
        for c in range(1, n_chunks):
            b, pb = c % 2, (c - 1) % 2
            pltpu.sync_copy(idx_hbm.at[pl.ds(base + c * ch, ch)], ibuf[b])
            nxt = pltpu.async_copy(table_hbm.at[ibuf[b]], rbuf[b], sem[b])
            pending.wait()
            pltpu.sync_copy(rbuf[pb], out_hbm.at[pl.ds(base + (c - 1) * ch, ch)])
            pending = nxt
        pending.wait()
        last = n_chunks - 1
        pltpu.sync_copy(rbuf[last % 2], out_hbm.at[pl.ds(base + last * ch, ch)])

    return k


def kernel(embeddings, codebooks):
    B_, T_, D_ = embeddings.shape
    n_cb, K, _ = codebooks.shape
    N = B_ * T_
    rows = 1200
    G = 1
    NH = N // G
    NPADH = ((NH + 1023) // 1024) * 1024
    gather = _make_sc_gather(K, D_, NPADH)
    residual = embeddings
    qs = []
    codes_l = []
    total_loss = jnp.float32(0.0)
    for l in range(n_cb):
        w = codebooks[l]
        b2 = jnp.sum(w * w, axis=1).reshape(1, K)
        a2f = jnp.sum(residual * residual, axis=2).reshape(N, 1)
        r2d = residual.reshape(N, D_)
        idx_g, q_g = [], []
        for g in range(G):
            sl = slice(g * NH, (g + 1) * NH)
            rg = r2d if G == 1 else r2d[sl]
            ag = a2f if G == 1 else a2f[sl]
            idx = _level_indices(rg, w, b2, ag, rows=rows)
            idx_g.append(idx.reshape(NH))
            idxp = jnp.pad(idx_g[g], (0, NPADH - NH))
            q_g.append(gather(w, idxp)[:NH])
        codes_l.append(idx_g[0] if G == 1 else jnp.concatenate(idx_g, axis=0))
        qfull = (q_g[0] if G == 1
                 else jnp.concatenate(q_g, axis=0)).reshape(B_, T_, D_)
        qs.append(qfull)
        total_loss = total_loss + jnp.mean(
            (jax.lax.stop_gradient(residual) - qfull) ** 2)
        residual = residual - jax.lax.stop_gradient(qfull)
    codes = jnp.stack(codes_l, axis=-1).reshape(B_, T_, n_cb)
    quantized = qs[0]
    for l in range(1, n_cb):
        quantized = quantized + qs[l]
    quantized = embeddings + jax.lax.stop_gradient(quantized - embeddings)
    return codes, quantized, total_loss / n_cb


# speedup vs baseline: 1.3262x; 1.0668x over previous
import functools

import jax
import jax.numpy as jnp
from jax import lax
from jax.experimental import pallas as pl
from jax.experimental.pallas import tpu as pltpu
from jax.experimental.pallas import tpu_sc as plsc

_PH = (768, 1536)


def _level_body(resid_ref, w_ref, b2_ref, a2_ref, idx_ref, *, K):
    resid = resid_ref[...]
    w = w_ref[...]
    rb = resid.astype(jnp.bfloat16)
    wb = w.astype(jnp.bfloat16)
    conv = jax.lax.dot_general(
        rb, wb, (((1,), (1,)), ((), ())),
        preferred_element_type=jnp.float32)
    a2 = a2_ref[...]
    b2 = b2_ref[...]
    s = jnp.maximum((a2 + b2) - 2.0 * conv, 0.0)
    dd = s * jax.lax.rsqrt(s)
    R = resid.shape[0]
    iota = jax.lax.broadcasted_iota(jnp.int32, (R, K), 1)
    acc_v = None
    acc_i = None
    for lo, hi in ((0, _PH[0]), (_PH[0], _PH[1]), (_PH[1], K)):
        seg = dd[:, lo:hi]
        m = jnp.min(seg, axis=1, keepdims=True)
        si = jnp.min(jnp.where(seg == m, iota[:, lo:hi], K),
                     axis=1, keepdims=True)
        mb = m.astype(jnp.bfloat16).astype(jnp.float32)
        if acc_v is None:
            acc_v, acc_i = mb, si
        else:
            upd = m < acc_v
            acc_i = jnp.where(upd, si, acc_i)
            acc_v = jnp.where(upd, mb, acc_v)
    idx_ref[...] = jnp.clip(acc_i, 0, K - 1)


def _level_indices(resid, w, b2, a2, *, rows, npad):
    N, D = resid.shape
    K = w.shape[0]
    grid = (npad // rows,)
    idx = pl.pallas_call(
        functools.partial(_level_body, K=K),
        grid=grid,
        in_specs=[
            pl.BlockSpec((rows, D), lambda i: (i, 0)),
            pl.BlockSpec((K, D), lambda i: (0, 0)),
            pl.BlockSpec((1, K), lambda i: (0, 0)),
            pl.BlockSpec((rows, 1), lambda i: (i, 0)),
        ],
        out_specs=pl.BlockSpec((rows, 1), lambda i: (i, 0)),
        out_shape=jax.ShapeDtypeStruct((npad, 1), jnp.int32),
        compiler_params=pltpu.CompilerParams(
            dimension_semantics=("arbitrary",)),
    )(resid, w, b2, a2)
    return idx


def _make_sc_gather(V, D, B):
    info = plsc.get_sparse_core_info()
    NC, NS = info.num_cores, info.num_subcores
    NW = NC * NS
    b_per_w = B // NW
    n_chunks = max(1, b_per_w // 96)
    ch = b_per_w // n_chunks
    mesh = plsc.VectorSubcoreMesh(core_axis_name="c", subcore_axis_name="s")

    @functools.partial(
        pl.kernel, mesh=mesh,
        out_type=jax.ShapeDtypeStruct((B, D), jnp.float32),
        scratch_types=[
            pltpu.VMEM((ch,), jnp.int32),
            pltpu.VMEM((ch,), jnp.int32),
            pltpu.VMEM((ch, D), jnp.float32),
            pltpu.VMEM((ch, D), jnp.float32),
            pltpu.SemaphoreType.DMA,
            pltpu.SemaphoreType.DMA,
        ],
    )
    def k(table_hbm, idx_hbm, out_hbm, i0, i1, r0, r1, s0, s1):
        wid = lax.axis_index("s") * NC + lax.axis_index("c")
        base = wid * b_per_w
        ibuf, rbuf, sem = (i0, i1), (r0, r1), (s0, s1)
        pltpu.sync_copy(idx_hbm.at[pl.ds(base, ch)], i0)
        pending = pltpu.async_copy(table_hbm.at[i0], r0, s0)
        for c in range(1, n_chunks):
            b, pb = c % 2, (c - 1) % 2
            pltpu.sync_copy(idx_hbm.at[pl.ds(base + c * ch, ch)], ibuf[b])
            nxt = pltpu.async_copy(table_hbm.at[ibuf[b]], rbuf[b], sem[b])
            pending.wait()
            pltpu.sync_copy(rbuf[pb], out_hbm.at[pl.ds(base + (c - 1) * ch, ch)])
            pending = nxt
        pending.wait()
        last = n_chunks - 1
        pltpu.sync_copy(rbuf[last % 2], out_hbm.at[pl.ds(base + last * ch, ch)])

    return k


def kernel(embeddings, codebooks):
    B_, T_, D_ = embeddings.shape
    n_cb, K, _ = codebooks.shape
    N = B_ * T_
    rows = 1024
    NPADH = ((N + 1023) // 1024) * 1024
    gather = _make_sc_gather(K, D_, NPADH)
    residual = embeddings
    qs = []
    codes_l = []
    total_loss = jnp.float32(0.0)
    for l in range(n_cb):
        w = codebooks[l]
        b2 = jnp.sum(w * w, axis=1).reshape(1, K)
        a2f = jnp.sum(residual * residual, axis=2).reshape(N, 1)
        r2d = residual.reshape(N, D_)
        idxp = _level_indices(r2d, w, b2, a2f, rows=rows, npad=NPADH)
        idx1 = idxp.reshape(NPADH)
        codes_l.append(idx1[:N])
        qfull = gather(w, idx1)[:N].reshape(B_, T_, D_)
        qs.append(qfull)
        total_loss = total_loss + jnp.mean(
            (jax.lax.stop_gradient(residual) - qfull) ** 2)
        residual = residual - jax.lax.stop_gradient(qfull)
    codes = jnp.stack(codes_l, axis=-1).reshape(B_, T_, n_cb)
    quantized = qs[0]
    for l in range(1, n_cb):
        quantized = quantized + qs[l]
    quantized = embeddings + jax.lax.stop_gradient(quantized - embeddings)
    return codes, quantized, total_loss / n_cb


# speedup vs baseline: 1.3707x; 1.0336x over previous
import functools

import jax
import jax.numpy as jnp
from jax import lax
from jax.experimental import pallas as pl
from jax.experimental.pallas import tpu as pltpu
from jax.experimental.pallas import tpu_sc as plsc

_PH = (768, 1536)


def _level_body(resid_ref, w_ref, b2_ref, a2_ref, idx_ref, *, K):
    resid = resid_ref[...]
    w = w_ref[...]
    rb = resid.astype(jnp.bfloat16)
    wb = w.astype(jnp.bfloat16)
    conv = jax.lax.dot_general(
        rb, wb, (((1,), (1,)), ((), ())),
        preferred_element_type=jnp.float32)
    a2 = a2_ref[...]
    b2 = b2_ref[...]
    s = jnp.maximum((a2 + b2) - 2.0 * conv, 0.0)
    dd = s * jax.lax.rsqrt(s)
    R = resid.shape[0]
    iota = jax.lax.broadcasted_iota(jnp.int32, (R, K), 1)
    acc_v = None
    acc_i = None
    for lo, hi in ((0, _PH[0]), (_PH[0], _PH[1]), (_PH[1], K)):
        seg = dd[:, lo:hi]
        m = jnp.min(seg, axis=1, keepdims=True)
        si = jnp.min(jnp.where(seg == m, iota[:, lo:hi], K),
                     axis=1, keepdims=True)
        mb = m.astype(jnp.bfloat16).astype(jnp.float32)
        if acc_v is None:
            acc_v, acc_i = mb, si
        else:
            upd = m < acc_v
            acc_i = jnp.where(upd, si, acc_i)
            acc_v = jnp.where(upd, mb, acc_v)
    idx_ref[...] = jnp.clip(acc_i, 0, K - 1)


def _level_indices(resid, w, b2, a2, *, rows, npad):
    N, D = resid.shape
    K = w.shape[0]
    grid = (npad // rows,)
    idx = pl.pallas_call(
        functools.partial(_level_body, K=K),
        grid=grid,
        in_specs=[
            pl.BlockSpec((rows, D), lambda i: (i, 0)),
            pl.BlockSpec((K, D), lambda i: (0, 0)),
            pl.BlockSpec((1, K), lambda i: (0, 0)),
            pl.BlockSpec((rows, 1), lambda i: (i, 0)),
        ],
        out_specs=pl.BlockSpec((rows, 1), lambda i: (i, 0)),
        out_shape=jax.ShapeDtypeStruct((npad, 1), jnp.int32),
        compiler_params=pltpu.CompilerParams(
            dimension_semantics=("arbitrary",)),
    )(resid, w, b2, a2)
    return idx


def _make_sc_gather(V, D, B):
    info = plsc.get_sparse_core_info()
    NC, NS = info.num_cores, info.num_subcores
    NW = NC * NS
    b_per_w = B // NW
    n_chunks = max(1, b_per_w // 96)
    ch = b_per_w // n_chunks
    mesh = plsc.VectorSubcoreMesh(core_axis_name="c", subcore_axis_name="s")

    @functools.partial(
        pl.kernel, mesh=mesh,
        out_type=jax.ShapeDtypeStruct((B, D), jnp.float32),
        scratch_types=[
            pltpu.VMEM((ch,), jnp.int32),
            pltpu.VMEM((ch,), jnp.int32),
            pltpu.VMEM((ch, D), jnp.float32),
            pltpu.VMEM((ch, D), jnp.float32),
            pltpu.SemaphoreType.DMA,
            pltpu.SemaphoreType.DMA,
        ],
    )
    def k(table_hbm, idx_hbm, out_hbm, i0, i1, r0, r1, s0, s1):
        wid = lax.axis_index("s") * NC + lax.axis_index("c")
        base = wid * b_per_w
        ibuf, rbuf, sem = (i0, i1), (r0, r1), (s0, s1)
        pltpu.sync_copy(idx_hbm.at[pl.ds(base, ch)], i0)
        pending = pltpu.async_copy(table_hbm.at[i0], r0, s0)
        for c in range(1, n_chunks):
            b, pb = c % 2, (c - 1) % 2
            pltpu.sync_copy(idx_hbm.at[pl.ds(base + c * ch, ch)], ibuf[b])
            nxt = pltpu.async_copy(table_hbm.at[ibuf[b]], rbuf[b], sem[b])
            pending.wait()
            pltpu.sync_copy(rbuf[pb], out_hbm.at[pl.ds(base + (c - 1) * ch, ch)])
            pending = nxt
        pending.wait()
        last = n_chunks - 1
        pltpu.sync_copy(rbuf[last % 2], out_hbm.at[pl.ds(base + last * ch, ch)])

    return k


def kernel(embeddings, codebooks):
    B_, T_, D_ = embeddings.shape
    n_cb, K, _ = codebooks.shape
    N = B_ * T_
    rows = 1024
    NPADH = ((N + 1023) // 1024) * 1024
    gather = _make_sc_gather(K, D_, NPADH)
    residual = embeddings
    codes_l = []
    total_loss = jnp.float32(0.0)
    for l in range(n_cb):
        w = codebooks[l]
        b2 = jnp.sum(w * w, axis=1).reshape(1, K)
        a2f = jnp.sum(residual * residual, axis=2).reshape(N, 1)
        r2d = residual.reshape(N, D_)
        idxp = _level_indices(r2d, w, b2, a2f, rows=rows, npad=NPADH)
        idx1 = idxp.reshape(NPADH)
        codes_l.append(idx1[:N])
        qfull = gather(w, idx1)[:N].reshape(B_, T_, D_)
        residual = residual - jax.lax.stop_gradient(qfull)
        total_loss = total_loss + jnp.mean(residual * residual)
    codes = jnp.stack(codes_l, axis=-1).reshape(B_, T_, n_cb)
    quantized = embeddings - jax.lax.stop_gradient(residual)
    return codes, quantized, total_loss / n_cb
